# initial kernel scaffold (unmeasured)
import jax
import jax.numpy as jnp
from jax import lax
from jax.experimental import pallas as pl
from jax.experimental.pallas import tpu as pltpu


def kernel(
    x,
):
    def body(*refs):
        pass

    out_shape = jax.ShapeDtypeStruct(..., jnp.float32)
    return pl.pallas_call(body, out_shape=out_shape)(...)


# baseline (device time: 15584 ns/iter reference)
import jax
import jax.numpy as jnp
from jax import lax
from jax.experimental import pallas as pl
from jax.experimental.pallas import tpu as pltpu


def kernel(x):
    m, n_local = x.shape
    n_global = 2 * n_local

    def body(x_ref, out_ref, acc_ref, comm_ref, send_sem, recv_sem):
        my_x = lax.axis_index("x")
        my_y = lax.axis_index("y")
        peer = (my_x, 1 - my_y)

        barrier_sem = pltpu.get_barrier_semaphore()
        pl.semaphore_signal(
            barrier_sem, inc=1, device_id=peer,
            device_id_type=pl.DeviceIdType.MESH,
        )
        pl.semaphore_wait(barrier_sem, 1)

        acc_ref[:, :] = jnp.sum(x_ref[:, :], axis=1, keepdims=True)

        rdma = pltpu.make_async_remote_copy(
            src_ref=acc_ref,
            dst_ref=comm_ref,
            send_sem=send_sem,
            recv_sem=recv_sem,
            device_id=peer,
            device_id_type=pl.DeviceIdType.MESH,
        )
        rdma.start()
        rdma.wait()

        out_ref[:, :] = (acc_ref[:, :] + comm_ref[:, :]) * (1.0 / n_global)

    return pl.pallas_call(
        body,
        out_shape=jax.ShapeDtypeStruct((m, 1), jnp.float32),
        in_specs=[pl.BlockSpec(memory_space=pltpu.VMEM)],
        out_specs=pl.BlockSpec(memory_space=pltpu.VMEM),
        scratch_shapes=[
            pltpu.VMEM((m, 1), jnp.float32),
            pltpu.VMEM((m, 1), jnp.float32),
            pltpu.SemaphoreType.DMA,
            pltpu.SemaphoreType.DMA,
        ],
        compiler_params=pltpu.CompilerParams(collective_id=0),
    )(x)


# device time: 6847 ns/iter; 2.2760x vs baseline; 2.2760x over previous
import jax
import jax.numpy as jnp
from jax import lax
from jax.experimental import pallas as pl
from jax.experimental.pallas import tpu as pltpu


def kernel(x):
    m, n_local = x.shape
    n_global = 2 * n_local

    pk_rows = m // 128

    def body(x_ref, out_ref, acc_ref, comm_ref, send_sem, recv_sem):
        my_x = lax.axis_index("x")
        my_y = lax.axis_index("y")
        peer = (my_x, 1 - my_y)

        barrier_sem = pltpu.get_barrier_semaphore()
        pl.semaphore_signal(
            barrier_sem, inc=1, device_id=peer,
            device_id_type=pl.DeviceIdType.MESH,
        )
        pl.semaphore_wait(barrier_sem, 1)

        partial = jnp.sum(x_ref[:, :], axis=1, keepdims=True)
        acc_ref[:, :] = partial.reshape(pk_rows, 128)

        rdma = pltpu.make_async_remote_copy(
            src_ref=acc_ref,
            dst_ref=comm_ref,
            send_sem=send_sem,
            recv_sem=recv_sem,
            device_id=peer,
            device_id_type=pl.DeviceIdType.MESH,
        )
        rdma.start()
        rdma.wait()

        out_ref[:, :] = (acc_ref[:, :] + comm_ref[:, :]) * (1.0 / n_global)

    packed = pl.pallas_call(
        body,
        out_shape=jax.ShapeDtypeStruct((pk_rows, 128), jnp.float32),
        in_specs=[pl.BlockSpec(memory_space=pltpu.VMEM)],
        out_specs=pl.BlockSpec(memory_space=pltpu.VMEM),
        scratch_shapes=[
            pltpu.VMEM((pk_rows, 128), jnp.float32),
            pltpu.VMEM((pk_rows, 128), jnp.float32),
            pltpu.SemaphoreType.DMA,
            pltpu.SemaphoreType.DMA,
        ],
        compiler_params=pltpu.CompilerParams(collective_id=0),
    )(x)
    return packed.reshape(m, 1)


# device time: 6754 ns/iter; 2.3074x vs baseline; 1.0138x over previous
import jax
import jax.numpy as jnp
from jax import lax
from jax.experimental import pallas as pl
from jax.experimental.pallas import tpu as pltpu

BLK = 256


def kernel(x):
    m, n_local = x.shape
    n_global = 2 * n_local
    pk_rows = m // 128
    n_chunks = m // BLK
    pk_blk = BLK // 128
    send_at = {n_chunks // 2 - 1: 0, n_chunks - 1: 1}
    split = (n_chunks // 2) * pk_blk
    half_bounds = {0: (0, split), 1: (split, pk_rows)}

    def body(x_ref, out_ref, buf_ref, acc_ref, comm_ref,
             copy_sems, send_sems, recv_sems):
        my_x = lax.axis_index("x")
        my_y = lax.axis_index("y")
        peer = (my_x, 1 - my_y)

        barrier_sem = pltpu.get_barrier_semaphore()
        pl.semaphore_signal(
            barrier_sem, inc=1, device_id=peer,
            device_id_type=pl.DeviceIdType.MESH,
        )

        copies = [
            pltpu.make_async_copy(
                x_ref.at[pl.ds(k * BLK, BLK), :],
                buf_ref.at[k],
                copy_sems.at[k],
            )
            for k in range(n_chunks)
        ]
        for c in copies:
            c.start()

        rdmas = []
        for k, c in enumerate(copies):
            c.wait()
            partial = jnp.sum(buf_ref[k], axis=1, keepdims=True)
            acc_ref[pl.ds(k * pk_blk, pk_blk), :] = partial.reshape(pk_blk, 128)
            if k in send_at:
                h = send_at[k]
                if h == 0:
                    pl.semaphore_wait(barrier_sem, 1)
                lo, hi = half_bounds[h]
                rdma = pltpu.make_async_remote_copy(
                    src_ref=acc_ref.at[pl.ds(lo, hi - lo), :],
                    dst_ref=comm_ref.at[pl.ds(lo, hi - lo), :],
                    send_sem=send_sems.at[h],
                    recv_sem=recv_sems.at[h],
                    device_id=peer,
                    device_id_type=pl.DeviceIdType.MESH,
                )
                rdma.start()
                rdmas.append(rdma)

        for rdma in rdmas:
            rdma.wait()
        out_ref[:, :] = (acc_ref[:, :] + comm_ref[:, :]) * (1.0 / n_global)

    x = pltpu.with_memory_space_constraint(x, pltpu.MemorySpace.HBM)

    packed = pl.pallas_call(
        body,
        out_shape=jax.ShapeDtypeStruct((pk_rows, 128), jnp.float32),
        in_specs=[pl.BlockSpec(memory_space=pltpu.MemorySpace.HBM)],
        out_specs=pl.BlockSpec(memory_space=pltpu.MemorySpace.VMEM),
        scratch_shapes=[
            pltpu.VMEM((n_chunks, BLK, n_local), jnp.float32),
            pltpu.VMEM((pk_rows, 128), jnp.float32),
            pltpu.VMEM((pk_rows, 128), jnp.float32),
            pltpu.SemaphoreType.DMA((n_chunks,)),
            pltpu.SemaphoreType.DMA((2,)),
            pltpu.SemaphoreType.DMA((2,)),
        ],
        compiler_params=pltpu.CompilerParams(collective_id=0),
    )(x)
    return packed.reshape(m, 1)
